# baseline (speedup 1.0000x reference)
import functools

import jax
import jax.numpy as jnp
from jax import lax
from jax.experimental import pallas as pl
from jax.experimental.pallas import tpu as pltpu
from jax.experimental.pallas import tpu_sc as plsc

V = 100000
D = 512
L = 16
NC = 2
NS = 16
NW = NC * NS
SBR = 80
NSB = V // SBR
SBQ = NSB // NW
SBX = NSB - NW * SBQ


def _sc_topk_call(psi_flat, map_flat):
    mesh = plsc.VectorSubcoreMesh(core_axis_name="c", subcore_axis_name="s")

    @functools.partial(
        pl.kernel,
        mesh=mesh,
        out_type=[
            jax.ShapeDtypeStruct((NW, L), jnp.float32),
            jax.ShapeDtypeStruct((NW, L), jnp.float32),
            jax.ShapeDtypeStruct((NW, L), jnp.int32),
        ],
        scratch_types=[
            pltpu.VMEM((D,), jnp.float32),
            pltpu.VMEM((SBR * D,), jnp.float32),
            pltpu.VMEM((SBR * D,), jnp.float32),
            pltpu.VMEM((L * 2 * L,), jnp.float32),
            pltpu.VMEM((L,), jnp.float32),
            pltpu.VMEM((L,), jnp.float32),
            pltpu.VMEM((L,), jnp.int32),
            pltpu.SemaphoreType.DMA,
            pltpu.SemaphoreType.DMA,
        ],
    )
    def k(psi_hbm, map_hbm, out_num, out_den, out_idx, psi_v, buf0, buf1,
          f_v, rn_v, rd_v, ri_v, sem0, sem1):
        wid = lax.axis_index("c") * NS + lax.axis_index("s")
        nsb = jnp.where(wid < SBX, SBQ + 1, SBQ)
        sb0 = wid * SBQ + jnp.minimum(wid, SBX)

        pltpu.sync_copy(psi_hbm, psi_v)
        psi_regs = [psi_v[pl.ds(L * c, L)] for c in range(D // L)]
        zero = jnp.zeros((L,), jnp.float32)
        for r in range(L):
            f_v[pl.ds(r * 2 * L + L, L)] = zero

        def start(sb, buf, sem):
            pltpu.async_copy(map_hbm.at[pl.ds(sb * (SBR * D), SBR * D)],
                             buf, sem)

        def wait(buf, sem):
            pltpu.make_async_copy(map_hbm.at[pl.ds(0, SBR * D)],
                                  buf, sem).wait()

        def fold16(vec, r):
            base = r * 2 * L
            f_v[pl.ds(base, L)] = vec
            z = vec + f_v[pl.ds(base + 8, L)]
            f_v[pl.ds(base, L)] = z
            z = z + f_v[pl.ds(base + 4, L)]
            f_v[pl.ds(base, L)] = z
            z = z + f_v[pl.ds(base + 2, L)]
            f_v[pl.ds(base, L)] = z
            z = z + f_v[pl.ds(base + 1, L)]
            return z[0]

        def compute(buf, row0, carry):
            def sub(s, c2):
                bn, bd, bi = c2
                soff = s * (L * D)
                for r in range(L):
                    accn = zero
                    accs = zero
                    for c in range(D // L):
                        x = buf[pl.ds(soff + r * D + c * L, L)]
                        accn = accn + x * psi_regs[c]
                        accs = accs + x * x
                    num = fold16(accn, r)
                    ssq = jnp.maximum(fold16(accs, r), jnp.float32(1e-16))
                    n2 = num * jnp.where(num < 0.0, -num, num)
                    upd = n2 * bd > bn * ssq
                    bn = jnp.where(upd, n2, bn)
                    bd = jnp.where(upd, ssq, bd)
                    bi = jnp.where(upd, row0 + s * L + r, bi)
                return bn, bd, bi

            return lax.fori_loop(0, SBR // L, sub, carry)

        start(sb0, buf0, sem0)

        def step(i, carry):
            def even_branch(c2):
                wait(buf0, sem0)

                @pl.when(i + 1 < nsb)
                def _():
                    start(sb0 + i + 1, buf1, sem1)

                return compute(buf0, (sb0 + i) * SBR, c2)

            def odd_branch(c2):
                wait(buf1, sem1)

                @pl.when(i + 1 < nsb)
                def _():
                    start(sb0 + i + 1, buf0, sem0)

                return compute(buf1, (sb0 + i) * SBR, c2)

            return lax.cond(lax.rem(i, 2) == 0, even_branch, odd_branch,
                            carry)

        init = (jnp.float32(-3.4e38), jnp.float32(1.0), jnp.int32(0))
        bn, bd, bi = lax.fori_loop(0, nsb, step, init)
        rn_v[...] = jnp.full((L,), 1.0, jnp.float32) * bn
        rd_v[...] = jnp.full((L,), 1.0, jnp.float32) * bd
        ri_v[...] = jnp.full((L,), 1, jnp.int32) * bi
        pltpu.sync_copy(rn_v, out_num.at[wid])
        pltpu.sync_copy(rd_v, out_den.at[wid])
        pltpu.sync_copy(ri_v, out_idx.at[wid])

    return k(psi_flat, map_flat)


def kernel(psi_final, spectral_map):
    psi_flat = psi_final.reshape(-1)
    map_flat = spectral_map.reshape(-1)
    nums, dens, idxs = _sc_topk_call(psi_flat, map_flat)
    n = nums[:, 0]
    d = dens[:, 0]
    ix = idxs[:, 0]
    cross = n[:, None] * d[None, :]
    strictly = cross.T > cross
    tie = (cross.T == cross) & (ix[:, None] > ix[None, :])
    loses = jnp.any(strictly | tie, axis=1)
    best = jnp.min(jnp.where(loses, jnp.int32(V), ix))
    return best.astype(jnp.int32)


# speedup vs baseline: 13.6030x; 13.6030x over previous
import functools

import jax
import jax.numpy as jnp
from jax import lax
from jax.experimental import pallas as pl
from jax.experimental.pallas import tpu as pltpu
from jax.experimental.pallas import tpu_sc as plsc

V = 100000
D = 512
L = 16
NC = 2
NS = 16
NW = NC * NS
SBR = 80
NSB = V // SBR
SBQ = NSB // NW
SBX = NSB - NW * SBQ


def _sc_topk_call(psi_flat, map_flat):
    mesh = plsc.VectorSubcoreMesh(core_axis_name="c", subcore_axis_name="s")

    @functools.partial(
        pl.kernel,
        mesh=mesh,
        out_type=[
            jax.ShapeDtypeStruct((NW, L), jnp.float32),
            jax.ShapeDtypeStruct((NW, L), jnp.float32),
            jax.ShapeDtypeStruct((NW, L), jnp.int32),
        ],
        scratch_types=[
            pltpu.VMEM((D,), jnp.float32),
            pltpu.VMEM((SBR, D), jnp.float32),
            pltpu.VMEM((SBR, D), jnp.float32),
            pltpu.VMEM((L * 2 * L,), jnp.float32),
            pltpu.VMEM((L,), jnp.float32),
            pltpu.VMEM((L,), jnp.float32),
            pltpu.VMEM((L,), jnp.int32),
            pltpu.SemaphoreType.DMA,
            pltpu.SemaphoreType.DMA,
        ],
    )
    def k(psi_hbm, map_hbm, out_num, out_den, out_idx, psi_v, buf0, buf1,
          f_v, rn_v, rd_v, ri_v, sem0, sem1):
        wid = lax.axis_index("c") * NS + lax.axis_index("s")
        nsb = jnp.where(wid < SBX, SBQ + 1, SBQ)
        sb0 = wid * SBQ + jnp.minimum(wid, SBX)

        pltpu.sync_copy(psi_hbm, psi_v)
        psi_regs = [psi_v[pl.ds(L * c, L)] for c in range(D // L)]
        zero = jnp.zeros((L,), jnp.float32)
        for r in range(L):
            f_v[pl.ds(r * 2 * L + L, L)] = zero

        def start(sb, buf, sem):
            pltpu.async_copy(map_hbm.at[pl.ds(sb * SBR, SBR)], buf, sem)

        def wait(buf, sem):
            pltpu.make_async_copy(map_hbm.at[pl.ds(0, SBR)], buf,
                                  sem).wait()

        def fold16(vec, r):
            base = r * 2 * L
            f_v[pl.ds(base, L)] = vec
            z = vec + f_v[pl.ds(base + 8, L)]
            f_v[pl.ds(base, L)] = z
            z = z + f_v[pl.ds(base + 4, L)]
            f_v[pl.ds(base, L)] = z
            z = z + f_v[pl.ds(base + 2, L)]
            f_v[pl.ds(base, L)] = z
            z = z + f_v[pl.ds(base + 1, L)]
            return z[0]

        def compute(buf, row0, carry):
            def sub(s, c2):
                bn, bd, bi = c2
                srow = s * L
                for r in range(L):
                    accn = zero
                    accs = zero
                    for c in range(D // L):
                        x = buf[srow + r, pl.ds(c * L, L)]
                        accn = accn + x * psi_regs[c]
                        accs = accs + x * x
                    num = fold16(accn, r)
                    ssq = jnp.maximum(fold16(accs, r), jnp.float32(1e-16))
                    n2 = num * jnp.where(num < 0.0, -num, num)
                    upd = n2 * bd > bn * ssq
                    bn = jnp.where(upd, n2, bn)
                    bd = jnp.where(upd, ssq, bd)
                    bi = jnp.where(upd, row0 + s * L + r, bi)
                return bn, bd, bi

            return lax.fori_loop(0, SBR // L, sub, carry)

        start(sb0, buf0, sem0)

        def step(i, carry):
            def even_branch(c2):
                wait(buf0, sem0)

                @pl.when(i + 1 < nsb)
                def _():
                    start(sb0 + i + 1, buf1, sem1)

                return compute(buf0, (sb0 + i) * SBR, c2)

            def odd_branch(c2):
                wait(buf1, sem1)

                @pl.when(i + 1 < nsb)
                def _():
                    start(sb0 + i + 1, buf0, sem0)

                return compute(buf1, (sb0 + i) * SBR, c2)

            return lax.cond(lax.rem(i, 2) == 0, even_branch, odd_branch,
                            carry)

        init = (jnp.float32(-3.4e38), jnp.float32(1.0), jnp.int32(0))
        bn, bd, bi = lax.fori_loop(0, nsb, step, init)
        rn_v[...] = jnp.full((L,), 1.0, jnp.float32) * bn
        rd_v[...] = jnp.full((L,), 1.0, jnp.float32) * bd
        ri_v[...] = jnp.full((L,), 1, jnp.int32) * bi
        pltpu.sync_copy(rn_v, out_num.at[wid])
        pltpu.sync_copy(rd_v, out_den.at[wid])
        pltpu.sync_copy(ri_v, out_idx.at[wid])

    return k(psi_flat, map_flat)


def kernel(psi_final, spectral_map):
    psi_flat = psi_final.reshape(-1)
    map2d = spectral_map.reshape(V, D)
    nums, dens, idxs = _sc_topk_call(psi_flat, map2d)
    n = nums[:, 0]
    d = dens[:, 0]
    ix = idxs[:, 0]
    cross = n[:, None] * d[None, :]
    strictly = cross.T > cross
    tie = (cross.T == cross) & (ix[:, None] > ix[None, :])
    loses = jnp.any(strictly | tie, axis=1)
    best = jnp.min(jnp.where(loses, jnp.int32(V), ix))
    return best.astype(jnp.int32)


# speedup vs baseline: 19.8633x; 1.4602x over previous
import functools

import jax
import jax.numpy as jnp
from jax import lax
from jax.experimental import pallas as pl
from jax.experimental.pallas import tpu as pltpu
from jax.experimental.pallas import tpu_sc as plsc

V = 100000
D = 512
L = 16
NC = 2
NS = 16
NW = NC * NS
SBR = 80
NSB = V // SBR
SBQ = NSB // NW
SBX = NSB - NW * SBQ


def _sc_topk_call(psi_flat, map_flat):
    mesh = plsc.VectorSubcoreMesh(core_axis_name="c", subcore_axis_name="s")

    @functools.partial(
        pl.kernel,
        mesh=mesh,
        out_type=[
            jax.ShapeDtypeStruct((NW, L), jnp.float32),
            jax.ShapeDtypeStruct((NW, L), jnp.float32),
            jax.ShapeDtypeStruct((NW, L), jnp.int32),
        ],
        scratch_types=[
            pltpu.VMEM((D,), jnp.float32),
            pltpu.VMEM((SBR, D), jnp.float32),
            pltpu.VMEM((SBR, D), jnp.float32),
            pltpu.VMEM((L,), jnp.float32),
            pltpu.VMEM((L,), jnp.float32),
            pltpu.VMEM((L,), jnp.int32),
            pltpu.SemaphoreType.DMA,
            pltpu.SemaphoreType.DMA,
        ] + [pltpu.VMEM((4 * L,), jnp.float32) for _ in range(L)],
    )
    def k(psi_hbm, map_hbm, out_num, out_den, out_idx, psi_v, buf0, buf1,
          rn_v, rd_v, ri_v, sem0, sem1, *folds):
        wid = lax.axis_index("c") * NS + lax.axis_index("s")
        nsb = jnp.where(wid < SBX, SBQ + 1, SBQ)
        sb0 = wid * SBQ + jnp.minimum(wid, SBX)

        pltpu.sync_copy(psi_hbm, psi_v)
        psi_regs = [psi_v[pl.ds(L * c, L)] for c in range(D // L)]
        zero = jnp.zeros((L,), jnp.float32)
        for r in range(L):
            folds[r][pl.ds(L, L)] = zero
            folds[r][pl.ds(3 * L, L)] = zero

        def start(sb, buf, sem):
            pltpu.async_copy(map_hbm.at[pl.ds(sb * SBR, SBR)], buf, sem)

        def wait(buf, sem):
            pltpu.make_async_copy(map_hbm.at[pl.ds(0, SBR)], buf,
                                  sem).wait()

        def fold16(vec, r, half):
            f_v = folds[r]
            base = half * 2 * L
            f_v[pl.ds(base, L)] = vec
            z = vec + f_v[pl.ds(base + 8, L)]
            f_v[pl.ds(base, L)] = z
            z = z + f_v[pl.ds(base + 4, L)]
            return (z[0] + z[1]) + (z[2] + z[3])

        def compute(buf, row0, carry):
            def sub(s, c2):
                bn, bd, bi = c2
                srow = s * L
                for r in range(L):
                    accn = zero
                    accs = zero
                    for c in range(D // L):
                        x = buf[srow + r, pl.ds(c * L, L)]
                        accn = accn + x * psi_regs[c]
                        accs = accs + x * x
                    num = fold16(accn, r, 0)
                    ssq = jnp.maximum(fold16(accs, r, 1), jnp.float32(1e-16))
                    n2 = num * jnp.where(num < 0.0, -num, num)
                    upd = n2 * bd > bn * ssq
                    bn = jnp.where(upd, n2, bn)
                    bd = jnp.where(upd, ssq, bd)
                    bi = jnp.where(upd, row0 + s * L + r, bi)
                return bn, bd, bi

            return lax.fori_loop(0, SBR // L, sub, carry)

        start(sb0, buf0, sem0)

        def step(i, carry):
            def even_branch(c2):
                wait(buf0, sem0)

                @pl.when(i + 1 < nsb)
                def _():
                    start(sb0 + i + 1, buf1, sem1)

                return compute(buf0, (sb0 + i) * SBR, c2)

            def odd_branch(c2):
                wait(buf1, sem1)

                @pl.when(i + 1 < nsb)
                def _():
                    start(sb0 + i + 1, buf0, sem0)

                return compute(buf1, (sb0 + i) * SBR, c2)

            return lax.cond(lax.rem(i, 2) == 0, even_branch, odd_branch,
                            carry)

        init = (jnp.float32(-3.4e38), jnp.float32(1.0), jnp.int32(0))
        bn, bd, bi = lax.fori_loop(0, nsb, step, init)
        rn_v[...] = jnp.full((L,), 1.0, jnp.float32) * bn
        rd_v[...] = jnp.full((L,), 1.0, jnp.float32) * bd
        ri_v[...] = jnp.full((L,), 1, jnp.int32) * bi
        pltpu.sync_copy(rn_v, out_num.at[wid])
        pltpu.sync_copy(rd_v, out_den.at[wid])
        pltpu.sync_copy(ri_v, out_idx.at[wid])

    return k(psi_flat, map_flat)


def kernel(psi_final, spectral_map):
    psi_flat = psi_final.reshape(-1)
    map2d = spectral_map.reshape(V, D)
    nums, dens, idxs = _sc_topk_call(psi_flat, map2d)
    n = nums[:, 0]
    d = dens[:, 0]
    ix = idxs[:, 0]
    cross = n[:, None] * d[None, :]
    strictly = cross.T > cross
    tie = (cross.T == cross) & (ix[:, None] > ix[None, :])
    loses = jnp.any(strictly | tie, axis=1)
    best = jnp.min(jnp.where(loses, jnp.int32(V), ix))
    return best.astype(jnp.int32)


# speedup vs baseline: 23.7786x; 1.1971x over previous
import functools

import jax
import jax.numpy as jnp
from jax import lax
from jax.experimental import pallas as pl
from jax.experimental.pallas import tpu as pltpu
from jax.experimental.pallas import tpu_sc as plsc

V = 100000
D = 512
L = 16
NC = 2
NS = 16
NW = NC * NS
SBR = 80

V_SC = 25600
V_TC = V - V_SC
SBQ = V_SC // SBR // NW

BR = 800
NBLK = V_TC // BR


def _sc_topk_call(psi_pad, map2d):
    mesh = plsc.VectorSubcoreMesh(core_axis_name="c", subcore_axis_name="s")

    @functools.partial(
        pl.kernel,
        mesh=mesh,
        out_type=[
            jax.ShapeDtypeStruct((8, 128), jnp.float32),
            jax.ShapeDtypeStruct((8, 128), jnp.float32),
            jax.ShapeDtypeStruct((8, 128), jnp.int32),
        ],
        scratch_types=[
            pltpu.VMEM((8, 128), jnp.float32),
            pltpu.VMEM((SBR, D), jnp.float32),
            pltpu.VMEM((SBR, D), jnp.float32),
            pltpu.VMEM((L,), jnp.float32),
            pltpu.VMEM((L,), jnp.float32),
            pltpu.VMEM((L,), jnp.int32),
            pltpu.SemaphoreType.DMA,
            pltpu.SemaphoreType.DMA,
        ] + [pltpu.VMEM((4 * L,), jnp.float32) for _ in range(L)],
    )
    def k(psi_hbm, map_hbm, out_num, out_den, out_idx, psi_v, buf0, buf1,
          rn_v, rd_v, ri_v, sem0, sem1, *folds):
        wid = lax.axis_index("c") * NS + lax.axis_index("s")
        sb0 = wid * SBQ

        pltpu.sync_copy(psi_hbm, psi_v)
        psi_regs = [psi_v[c // 8, pl.ds((c % 8) * L, L)]
                    for c in range(D // L)]
        zero = jnp.zeros((L,), jnp.float32)
        for r in range(L):
            folds[r][pl.ds(L, L)] = zero
            folds[r][pl.ds(3 * L, L)] = zero

        def start(sb, buf, sem):
            pltpu.async_copy(map_hbm.at[pl.ds(V_TC + sb * SBR, SBR)], buf,
                             sem)

        def wait(buf, sem):
            pltpu.make_async_copy(map_hbm.at[pl.ds(0, SBR)], buf,
                                  sem).wait()

        def fold16(vec, r, half):
            f_v = folds[r]
            base = half * 2 * L
            f_v[pl.ds(base, L)] = vec
            z = vec + f_v[pl.ds(base + 8, L)]
            f_v[pl.ds(base, L)] = z
            z = z + f_v[pl.ds(base + 4, L)]
            return (z[0] + z[1]) + (z[2] + z[3])

        def compute(buf, row0, carry):
            def sub(s, c2):
                bn, bd, bi = c2
                srow = s * L
                for r in range(L):
                    accn = zero
                    accs = zero
                    for c in range(D // L):
                        x = buf[srow + r, pl.ds(c * L, L)]
                        accn = accn + x * psi_regs[c]
                        accs = accs + x * x
                    num = fold16(accn, r, 0)
                    ssq = jnp.maximum(fold16(accs, r, 1), jnp.float32(1e-16))
                    n2 = num * jnp.where(num < 0.0, -num, num)
                    upd = n2 * bd > bn * ssq
                    bn = jnp.where(upd, n2, bn)
                    bd = jnp.where(upd, ssq, bd)
                    bi = jnp.where(upd, row0 + s * L + r, bi)
                return bn, bd, bi

            return lax.fori_loop(0, SBR // L, sub, carry)

        start(sb0, buf0, sem0)

        def step(i, carry):
            def even_branch(c2):
                wait(buf0, sem0)

                @pl.when(i + 1 < SBQ)
                def _():
                    start(sb0 + i + 1, buf1, sem1)

                return compute(buf0, V_TC + (sb0 + i) * SBR, c2)

            def odd_branch(c2):
                wait(buf1, sem1)

                @pl.when(i + 1 < SBQ)
                def _():
                    start(sb0 + i + 1, buf0, sem0)

                return compute(buf1, V_TC + (sb0 + i) * SBR, c2)

            return lax.cond(lax.rem(i, 2) == 0, even_branch, odd_branch,
                            carry)

        init = (jnp.float32(-3.4e38), jnp.float32(1.0), jnp.int32(0))
        bn, bd, bi = lax.fori_loop(0, SBQ, step, init)
        rn_v[...] = jnp.full((L,), 1.0, jnp.float32) * bn
        rd_v[...] = jnp.full((L,), 1.0, jnp.float32) * bd
        ri_v[...] = jnp.full((L,), 1, jnp.int32) * bi
        row = wid // 8
        col = (wid % 8) * L
        pltpu.sync_copy(rn_v, out_num.at[row, pl.ds(col, L)])
        pltpu.sync_copy(rd_v, out_den.at[row, pl.ds(col, L)])
        pltpu.sync_copy(ri_v, out_idx.at[row, pl.ds(col, L)])

    return k(psi_pad, map2d)


def _tc_topk_call(psi_row, map2d):
    def body(psi_ref, blk_ref, on_ref, od_ref, oi_ref, bn_s, bd_s, bi_s):
        i = pl.program_id(0)
        blk = blk_ref[...]
        psi = psi_ref[0, :]
        num = jnp.dot(blk, psi, preferred_element_type=jnp.float32,
                      precision=lax.Precision.HIGHEST)
        ssq = jnp.maximum(jnp.sum(blk * blk, axis=1), jnp.float32(1e-16))
        n2 = num * jnp.abs(num)
        key = num / jnp.sqrt(ssq)
        bidx = jnp.argmax(key)
        sel = lax.broadcasted_iota(jnp.int32, (BR,), 0) == bidx
        kbn = jnp.sum(jnp.where(sel, n2, 0.0))
        kbd = jnp.sum(jnp.where(sel, ssq, 0.0))

        @pl.when(i == 0)
        def _():
            bn_s[0] = jnp.float32(-3.4e38)
            bd_s[0] = jnp.float32(1.0)
            bi_s[0] = jnp.int32(0)

        take = kbn * bd_s[0] > bn_s[0] * kbd

        @pl.when(take)
        def _():
            bn_s[0] = kbn
            bd_s[0] = kbd
            bi_s[0] = i * BR + bidx

        @pl.when(i == NBLK - 1)
        def _():
            on_ref[...] = jnp.full((8, 128), bn_s[0], jnp.float32)
            od_ref[...] = jnp.full((8, 128), bd_s[0], jnp.float32)
            oi_ref[...] = jnp.full((8, 128), bi_s[0], jnp.int32)

    return pl.pallas_call(
        body,
        grid=(NBLK,),
        in_specs=[
            pl.BlockSpec((1, D), lambda i: (0, 0)),
            pl.BlockSpec((BR, D), lambda i: (i, 0)),
        ],
        out_specs=[
            pl.BlockSpec((8, 128), lambda i: (0, 0)),
            pl.BlockSpec((8, 128), lambda i: (0, 0)),
            pl.BlockSpec((8, 128), lambda i: (0, 0)),
        ],
        out_shape=[
            jax.ShapeDtypeStruct((8, 128), jnp.float32),
            jax.ShapeDtypeStruct((8, 128), jnp.float32),
            jax.ShapeDtypeStruct((8, 128), jnp.int32),
        ],
        scratch_shapes=[
            pltpu.SMEM((1,), jnp.float32),
            pltpu.SMEM((1,), jnp.float32),
            pltpu.SMEM((1,), jnp.int32),
        ],
    )(psi_row, map2d)


def kernel(psi_final, spectral_map):
    psi_flat = psi_final.reshape(-1)
    map2d = spectral_map.reshape(V, D)
    psi_pad = jnp.concatenate(
        [psi_flat, jnp.zeros((512,), jnp.float32)]).reshape(8, 128)

    sc_n, sc_d, sc_i = _sc_topk_call(psi_pad, map2d)
    tc_n, tc_d, tc_i = _tc_topk_call(psi_flat.reshape(1, D), map2d)

    n = jnp.concatenate([sc_n.reshape(-1)[:: L][:NW], tc_n[0, :1]])
    d = jnp.concatenate([sc_d.reshape(-1)[:: L][:NW], tc_d[0, :1]])
    ix = jnp.concatenate([sc_i.reshape(-1)[:: L][:NW], tc_i[0, :1]])
    cross = n[:, None] * d[None, :]
    strictly = cross.T > cross
    tie = (cross.T == cross) & (ix[:, None] > ix[None, :])
    loses = jnp.any(strictly | tie, axis=1)
    best = jnp.min(jnp.where(loses, jnp.int32(V), ix))
    return best.astype(jnp.int32)


# speedup vs baseline: 44.9444x; 1.8901x over previous
import functools

import jax
import jax.numpy as jnp
from jax import lax
from jax.experimental import pallas as pl
from jax.experimental.pallas import tpu as pltpu
from jax.experimental.pallas import tpu_sc as plsc

V = 100000
D = 512
L = 16
NC = 2
NS = 16
NW = NC * NS
SBR = 80

V_SC = 25600
V_TC = V - V_SC
SBQ = V_SC // SBR // NW

BR = 800
NBLK = V_TC // BR


def _sc_topk_call(psi_pad, map128):
    mesh = plsc.VectorSubcoreMesh(core_axis_name="c", subcore_axis_name="s")

    @functools.partial(
        pl.kernel,
        mesh=mesh,
        out_type=[
            jax.ShapeDtypeStruct((8, 128), jnp.float32),
            jax.ShapeDtypeStruct((8, 128), jnp.float32),
            jax.ShapeDtypeStruct((8, 128), jnp.int32),
        ],
        scratch_types=[
            pltpu.VMEM((8, 128), jnp.float32),
            pltpu.VMEM((SBR * 4, 128), jnp.float32),
            pltpu.VMEM((SBR * 4, 128), jnp.float32),
            pltpu.VMEM((L,), jnp.float32),
            pltpu.VMEM((L,), jnp.float32),
            pltpu.VMEM((L,), jnp.int32),
            pltpu.SemaphoreType.DMA,
            pltpu.SemaphoreType.DMA,
        ] + [pltpu.VMEM((4 * L,), jnp.float32) for _ in range(L)],
    )
    def k(psi_hbm, map_hbm, out_num, out_den, out_idx, psi_v, buf0, buf1,
          rn_v, rd_v, ri_v, sem0, sem1, *folds):
        wid = lax.axis_index("c") * NS + lax.axis_index("s")
        sb0 = wid * SBQ

        pltpu.sync_copy(psi_hbm, psi_v)
        psi_regs = [psi_v[c // 8, pl.ds((c % 8) * L, L)]
                    for c in range(D // L)]
        zero = jnp.zeros((L,), jnp.float32)
        for r in range(L):
            folds[r][pl.ds(L, L)] = zero
            folds[r][pl.ds(3 * L, L)] = zero

        def start(sb, buf, sem):
            pltpu.async_copy(
                map_hbm.at[pl.ds((V_TC + sb * SBR) * 4, SBR * 4)], buf, sem)

        def wait(buf, sem):
            pltpu.make_async_copy(map_hbm.at[pl.ds(0, SBR * 4)], buf,
                                  sem).wait()

        def fold16(vec, r, half):
            f_v = folds[r]
            base = half * 2 * L
            f_v[pl.ds(base, L)] = vec
            z = vec + f_v[pl.ds(base + 8, L)]
            f_v[pl.ds(base, L)] = z
            z = z + f_v[pl.ds(base + 4, L)]
            return (z[0] + z[1]) + (z[2] + z[3])

        def compute(buf, row0, carry):
            def sub(s, c2):
                bn, bd, bi = c2
                srow = s * L
                for r in range(L):
                    accn = zero
                    accs = zero
                    for c in range(D // L):
                        x = buf[(srow + r) * 4 + c // 8,
                                pl.ds((c % 8) * L, L)]
                        accn = accn + x * psi_regs[c]
                        accs = accs + x * x
                    num = fold16(accn, r, 0)
                    ssq = jnp.maximum(fold16(accs, r, 1), jnp.float32(1e-16))
                    n2 = num * jnp.where(num < 0.0, -num, num)
                    upd = n2 * bd > bn * ssq
                    bn = jnp.where(upd, n2, bn)
                    bd = jnp.where(upd, ssq, bd)
                    bi = jnp.where(upd, row0 + s * L + r, bi)
                return bn, bd, bi

            return lax.fori_loop(0, SBR // L, sub, carry)

        start(sb0, buf0, sem0)

        def step(i, carry):
            def even_branch(c2):
                wait(buf0, sem0)

                @pl.when(i + 1 < SBQ)
                def _():
                    start(sb0 + i + 1, buf1, sem1)

                return compute(buf0, V_TC + (sb0 + i) * SBR, c2)

            def odd_branch(c2):
                wait(buf1, sem1)

                @pl.when(i + 1 < SBQ)
                def _():
                    start(sb0 + i + 1, buf0, sem0)

                return compute(buf1, V_TC + (sb0 + i) * SBR, c2)

            return lax.cond(lax.rem(i, 2) == 0, even_branch, odd_branch,
                            carry)

        init = (jnp.float32(-3.4e38), jnp.float32(1.0), jnp.int32(0))
        bn, bd, bi = lax.fori_loop(0, SBQ, step, init)
        rn_v[...] = jnp.full((L,), 1.0, jnp.float32) * bn
        rd_v[...] = jnp.full((L,), 1.0, jnp.float32) * bd
        ri_v[...] = jnp.full((L,), 1, jnp.int32) * bi
        row = wid // 8
        col = (wid % 8) * L
        pltpu.sync_copy(rn_v, out_num.at[row, pl.ds(col, L)])
        pltpu.sync_copy(rd_v, out_den.at[row, pl.ds(col, L)])
        pltpu.sync_copy(ri_v, out_idx.at[row, pl.ds(col, L)])

    return k(psi_pad, map128)


def _tc_topk_call(psi_q, map128):
    def body(psi_ref, blk_ref, on_ref, od_ref, oi_ref, bn_s, bd_s, bi_s):
        i = pl.program_id(0)
        blk = blk_ref[...].reshape(BR, 4, 128)
        psi = psi_ref[...][:4].reshape(1, 4, 128)
        num = jnp.sum(blk * psi, axis=(1, 2))
        ssq = jnp.maximum(jnp.sum(blk * blk, axis=(1, 2)),
                          jnp.float32(1e-16))
        n2 = num * jnp.abs(num)
        key = num / jnp.sqrt(ssq)
        bidx = jnp.argmax(key)
        sel = lax.broadcasted_iota(jnp.int32, (BR,), 0) == bidx
        kbn = jnp.sum(jnp.where(sel, n2, 0.0))
        kbd = jnp.sum(jnp.where(sel, ssq, 0.0))

        @pl.when(i == 0)
        def _():
            bn_s[0] = jnp.float32(-3.4e38)
            bd_s[0] = jnp.float32(1.0)
            bi_s[0] = jnp.int32(0)

        take = kbn * bd_s[0] > bn_s[0] * kbd

        @pl.when(take)
        def _():
            bn_s[0] = kbn
            bd_s[0] = kbd
            bi_s[0] = i * BR + bidx

        @pl.when(i == NBLK - 1)
        def _():
            on_ref[...] = jnp.full((8, 128), bn_s[0], jnp.float32)
            od_ref[...] = jnp.full((8, 128), bd_s[0], jnp.float32)
            oi_ref[...] = jnp.full((8, 128), bi_s[0], jnp.int32)

    return pl.pallas_call(
        body,
        grid=(NBLK,),
        in_specs=[
            pl.BlockSpec((8, 128), lambda i: (0, 0)),
            pl.BlockSpec((BR * 4, 128), lambda i: (i, 0)),
        ],
        out_specs=[
            pl.BlockSpec((8, 128), lambda i: (0, 0)),
            pl.BlockSpec((8, 128), lambda i: (0, 0)),
            pl.BlockSpec((8, 128), lambda i: (0, 0)),
        ],
        out_shape=[
            jax.ShapeDtypeStruct((8, 128), jnp.float32),
            jax.ShapeDtypeStruct((8, 128), jnp.float32),
            jax.ShapeDtypeStruct((8, 128), jnp.int32),
        ],
        scratch_shapes=[
            pltpu.SMEM((1,), jnp.float32),
            pltpu.SMEM((1,), jnp.float32),
            pltpu.SMEM((1,), jnp.int32),
        ],
    )(psi_q, map128)


def kernel(psi_final, spectral_map):
    map128 = spectral_map.transpose(0, 2, 1).reshape(4 * V, 128)
    psi_q = psi_final.transpose(1, 0)
    psi_pad = jnp.concatenate([psi_q, jnp.zeros((4, 128), jnp.float32)])

    sc_n, sc_d, sc_i = _sc_topk_call(psi_pad, map128)
    tc_n, tc_d, tc_i = _tc_topk_call(psi_pad, map128)

    n = jnp.concatenate([sc_n.reshape(-1)[:: L][:NW], tc_n[0, :1]])
    d = jnp.concatenate([sc_d.reshape(-1)[:: L][:NW], tc_d[0, :1]])
    ix = jnp.concatenate([sc_i.reshape(-1)[:: L][:NW], tc_i[0, :1]])
    cross = n[:, None] * d[None, :]
    strictly = cross.T > cross
    tie = (cross.T == cross) & (ix[:, None] > ix[None, :])
    loses = jnp.any(strictly | tie, axis=1)
    best = jnp.min(jnp.where(loses, jnp.int32(V), ix))
    return best.astype(jnp.int32)


# speedup vs baseline: 66.0546x; 1.4697x over previous
import functools

import jax
import jax.numpy as jnp
from jax import lax
from jax.experimental import pallas as pl
from jax.experimental.pallas import tpu as pltpu
from jax.experimental.pallas import tpu_sc as plsc

V = 100000
D = 512
L = 16
NC = 2
NS = 16
NW = NC * NS
SBR = 80

V_SC = 51200
V_TC = V - V_SC
SBQ = V_SC // SBR // NW

BR = 800
NBLK = V_TC // BR


def _sc_topk_call(psi_pad, map128):
    mesh = plsc.VectorSubcoreMesh(core_axis_name="c", subcore_axis_name="s")

    @functools.partial(
        pl.kernel,
        mesh=mesh,
        out_type=[
            jax.ShapeDtypeStruct((8, 128), jnp.float32),
            jax.ShapeDtypeStruct((8, 128), jnp.float32),
            jax.ShapeDtypeStruct((8, 128), jnp.int32),
        ],
        scratch_types=[
            pltpu.VMEM((8, 128), jnp.float32),
            pltpu.VMEM((SBR * 4, 128), jnp.float32),
            pltpu.VMEM((SBR * 4, 128), jnp.float32),
            pltpu.VMEM((L,), jnp.float32),
            pltpu.VMEM((L,), jnp.float32),
            pltpu.VMEM((L,), jnp.int32),
            pltpu.SemaphoreType.DMA,
            pltpu.SemaphoreType.DMA,
        ] + [pltpu.VMEM((4 * L,), jnp.float32) for _ in range(L)],
    )
    def k(psi_hbm, map_hbm, out_num, out_den, out_idx, psi_v, buf0, buf1,
          rn_v, rd_v, ri_v, sem0, sem1, *folds):
        wid = lax.axis_index("c") * NS + lax.axis_index("s")
        sb0 = wid * SBQ

        pltpu.sync_copy(psi_hbm, psi_v)
        psi_regs = [psi_v[c // 8, pl.ds((c % 8) * L, L)]
                    for c in range(D // L)]
        zero = jnp.zeros((L,), jnp.float32)
        for r in range(L):
            folds[r][pl.ds(L, L)] = zero
            folds[r][pl.ds(3 * L, L)] = zero

        def start(sb, buf, sem):
            pltpu.async_copy(
                map_hbm.at[pl.ds((V_TC + sb * SBR) * 4, SBR * 4)], buf, sem)

        def wait(buf, sem):
            pltpu.make_async_copy(map_hbm.at[pl.ds(0, SBR * 4)], buf,
                                  sem).wait()

        def fold16(vec, r, half):
            f_v = folds[r]
            base = half * 2 * L
            f_v[pl.ds(base, L)] = vec
            z = vec + f_v[pl.ds(base + 8, L)]
            f_v[pl.ds(base, L)] = z
            z = z + f_v[pl.ds(base + 4, L)]
            return (z[0] + z[1]) + (z[2] + z[3])

        def compute(buf, row0, carry):
            def sub(s, c2):
                bn, bd, bi = c2
                srow = s * L
                for r in range(L):
                    accn = zero
                    accs = zero
                    for c in range(D // L):
                        x = buf[(srow + r) * 4 + c // 8,
                                pl.ds((c % 8) * L, L)]
                        accn = accn + x * psi_regs[c]
                        accs = accs + x * x
                    num = fold16(accn, r, 0)
                    ssq = jnp.maximum(fold16(accs, r, 1), jnp.float32(1e-16))
                    n2 = num * jnp.where(num < 0.0, -num, num)
                    upd = n2 * bd > bn * ssq
                    bn = jnp.where(upd, n2, bn)
                    bd = jnp.where(upd, ssq, bd)
                    bi = jnp.where(upd, row0 + s * L + r, bi)
                return bn, bd, bi

            return lax.fori_loop(0, SBR // L, sub, carry)

        start(sb0, buf0, sem0)

        def step(i, carry):
            def even_branch(c2):
                wait(buf0, sem0)

                @pl.when(i + 1 < SBQ)
                def _():
                    start(sb0 + i + 1, buf1, sem1)

                return compute(buf0, V_TC + (sb0 + i) * SBR, c2)

            def odd_branch(c2):
                wait(buf1, sem1)

                @pl.when(i + 1 < SBQ)
                def _():
                    start(sb0 + i + 1, buf0, sem0)

                return compute(buf1, V_TC + (sb0 + i) * SBR, c2)

            return lax.cond(lax.rem(i, 2) == 0, even_branch, odd_branch,
                            carry)

        init = (jnp.float32(-3.4e38), jnp.float32(1.0), jnp.int32(0))
        bn, bd, bi = lax.fori_loop(0, SBQ, step, init)
        rn_v[...] = jnp.full((L,), 1.0, jnp.float32) * bn
        rd_v[...] = jnp.full((L,), 1.0, jnp.float32) * bd
        ri_v[...] = jnp.full((L,), 1, jnp.int32) * bi
        row = wid // 8
        col = (wid % 8) * L
        pltpu.sync_copy(rn_v, out_num.at[row, pl.ds(col, L)])
        pltpu.sync_copy(rd_v, out_den.at[row, pl.ds(col, L)])
        pltpu.sync_copy(ri_v, out_idx.at[row, pl.ds(col, L)])

    return k(psi_pad, map128)


def _tc_topk_call(psi_q, map128):
    def body(psi_ref, blk_ref, on_ref, od_ref, oi_ref, bn_s, bd_s, bi_s):
        i = pl.program_id(0)
        blk = blk_ref[...].reshape(BR, 4, 128)
        psi = psi_ref[...][:4].reshape(1, 4, 128)
        num = jnp.sum(blk * psi, axis=(1, 2))
        ssq = jnp.maximum(jnp.sum(blk * blk, axis=(1, 2)),
                          jnp.float32(1e-16))
        n2 = num * jnp.abs(num)
        key = num / jnp.sqrt(ssq)
        bidx = jnp.argmax(key)
        sel = lax.broadcasted_iota(jnp.int32, (BR,), 0) == bidx
        kbn = jnp.sum(jnp.where(sel, n2, 0.0))
        kbd = jnp.sum(jnp.where(sel, ssq, 0.0))

        @pl.when(i == 0)
        def _():
            bn_s[0] = jnp.float32(-3.4e38)
            bd_s[0] = jnp.float32(1.0)
            bi_s[0] = jnp.int32(0)

        take = kbn * bd_s[0] > bn_s[0] * kbd

        @pl.when(take)
        def _():
            bn_s[0] = kbn
            bd_s[0] = kbd
            bi_s[0] = i * BR + bidx

        @pl.when(i == NBLK - 1)
        def _():
            on_ref[...] = jnp.full((8, 128), bn_s[0], jnp.float32)
            od_ref[...] = jnp.full((8, 128), bd_s[0], jnp.float32)
            oi_ref[...] = jnp.full((8, 128), bi_s[0], jnp.int32)

    return pl.pallas_call(
        body,
        grid=(NBLK,),
        in_specs=[
            pl.BlockSpec((8, 128), lambda i: (0, 0)),
            pl.BlockSpec((BR * 4, 128), lambda i: (i, 0)),
        ],
        out_specs=[
            pl.BlockSpec((8, 128), lambda i: (0, 0)),
            pl.BlockSpec((8, 128), lambda i: (0, 0)),
            pl.BlockSpec((8, 128), lambda i: (0, 0)),
        ],
        out_shape=[
            jax.ShapeDtypeStruct((8, 128), jnp.float32),
            jax.ShapeDtypeStruct((8, 128), jnp.float32),
            jax.ShapeDtypeStruct((8, 128), jnp.int32),
        ],
        scratch_shapes=[
            pltpu.SMEM((1,), jnp.float32),
            pltpu.SMEM((1,), jnp.float32),
            pltpu.SMEM((1,), jnp.int32),
        ],
    )(psi_q, map128)


def kernel(psi_final, spectral_map):
    map128 = spectral_map.transpose(0, 2, 1).reshape(4 * V, 128)
    psi_q = psi_final.transpose(1, 0)
    psi_pad = jnp.concatenate([psi_q, jnp.zeros((4, 128), jnp.float32)])

    sc_n, sc_d, sc_i = _sc_topk_call(psi_pad, map128)
    tc_n, tc_d, tc_i = _tc_topk_call(psi_pad, map128)

    n = jnp.concatenate([sc_n.reshape(-1)[:: L][:NW], tc_n[0, :1]])
    d = jnp.concatenate([sc_d.reshape(-1)[:: L][:NW], tc_d[0, :1]])
    ix = jnp.concatenate([sc_i.reshape(-1)[:: L][:NW], tc_i[0, :1]])
    cross = n[:, None] * d[None, :]
    strictly = cross.T > cross
    tie = (cross.T == cross) & (ix[:, None] > ix[None, :])
    loses = jnp.any(strictly | tie, axis=1)
    best = jnp.min(jnp.where(loses, jnp.int32(V), ix))
    return best.astype(jnp.int32)
